# baseline (speedup 1.0000x reference)
import functools

import jax
import jax.numpy as jnp
from jax.experimental import pallas as pl
from jax.experimental.pallas import tpu as pltpu

N = 10000
E = 160000
D_IN = 128
HID = 64
HEADS = 8
RULES = 10
OUT = 64
NEG = 0.2
EPS = 1e-5

NB = 1000


def _final_stage_kernel(h_ref, topo_ref, centers_ref, log_sigmas_ref,
                        rule_w_vec_ref, rule_W_ref, rule_b_ref, cls_W_ref,
                        cls_b_ref, out_ref, rules_ref):
    topo = topo_ref[...]
    c = centers_ref[...]
    q = 0.5 / (jnp.exp(log_sigmas_ref[...]) ** 2)
    A = jnp.dot(topo * topo, (q).T, preferred_element_type=jnp.float32)
    B = jnp.dot(topo, (c * q).T, preferred_element_type=jnp.float32)
    C = jnp.sum(c * c * q, axis=1)[None, :]
    logg = -(A - 2.0 * B + C)
    sig = 1.0 / (1.0 + jnp.exp(-rule_w_vec_ref[...]))
    rules = jnp.exp(logg) * sig
    rules_ref[...] = rules

    h = h_ref[...]
    rw = rule_W_ref[...]
    comb = (jnp.dot(h, rw[:HID], preferred_element_type=jnp.float32)
            + jnp.dot(rules, rw[HID:], preferred_element_type=jnp.float32)
            + rule_b_ref[...])
    h2 = jnp.maximum(comb, 0.0)
    o = jnp.dot(h2, cls_W_ref[...], preferred_element_type=jnp.float32) + cls_b_ref[...]
    m = jnp.max(o, axis=1, keepdims=True)
    lse = jnp.log(jnp.sum(jnp.exp(o - m), axis=1, keepdims=True)) + m
    out_ref[...] = o - lse


def _final_stage(h, topo, centers, log_sigmas, rule_weights, rule_W, rule_b,
                 cls_W, cls_b):
    grid = N // NB
    return pl.pallas_call(
        _final_stage_kernel,
        grid=(grid,),
        in_specs=[
            pl.BlockSpec((NB, HID), lambda i: (i, 0)),
            pl.BlockSpec((NB, 6), lambda i: (i, 0)),
            pl.BlockSpec((RULES, 6), lambda i: (0, 0)),
            pl.BlockSpec((RULES, 6), lambda i: (0, 0)),
            pl.BlockSpec((1, RULES), lambda i: (0, 0)),
            pl.BlockSpec((HID + RULES, HID), lambda i: (0, 0)),
            pl.BlockSpec((1, HID), lambda i: (0, 0)),
            pl.BlockSpec((HID, OUT), lambda i: (0, 0)),
            pl.BlockSpec((1, OUT), lambda i: (0, 0)),
        ],
        out_specs=[
            pl.BlockSpec((NB, OUT), lambda i: (i, 0)),
            pl.BlockSpec((NB, RULES), lambda i: (i, 0)),
        ],
        out_shape=[
            jax.ShapeDtypeStruct((N, OUT), jnp.float32),
            jax.ShapeDtypeStruct((N, RULES), jnp.float32),
        ],
    )(h, topo, centers, log_sigmas, rule_weights[None, :], rule_W,
      rule_b[None, :], cls_W, cls_b[None, :])


def _gat_conv(x, src, dst, W, a_src, a_dst, b, heads, out_ch):
    n = x.shape[0]
    h = (x @ W).reshape(n, heads, out_ch)
    alpha_src = (h * a_src[None]).sum(-1)
    alpha_dst = (h * a_dst[None]).sum(-1)
    alpha = alpha_src[src] + alpha_dst[dst]
    alpha = jnp.where(alpha >= 0.0, alpha, NEG * alpha)
    amax = jax.ops.segment_max(alpha, dst, num_segments=n)
    ex = jnp.exp(alpha - amax[dst])
    den = jax.ops.segment_sum(ex, dst, num_segments=n)
    att = ex / (den[dst] + 1e-16)
    out = jax.ops.segment_sum(h[src] * att[:, :, None], dst, num_segments=n)
    return out.reshape(n, heads * out_ch) + b


def kernel(x, edge_index, topo_features, W1, a_src1, a_dst1, b1, bn_g, bn_b,
           W2, a_src2, a_dst2, b2, centers, log_sigmas, rule_weights, rule_W,
           rule_b, cls_W, cls_b):
    n = x.shape[0]
    loop = jnp.arange(n)
    src = jnp.concatenate([edge_index[0], loop])
    dst = jnp.concatenate([edge_index[1], loop])
    h = _gat_conv(x, src, dst, W1, a_src1, a_dst1, b1, HEADS, HID)
    mu = h.mean(0)
    var = h.var(0)
    h = bn_g * (h - mu) / jnp.sqrt(var + EPS) + bn_b
    h = jax.nn.elu(h)
    h = _gat_conv(h, src, dst, W2, a_src2, a_dst2, b2, 1, HID)
    out, rules = _final_stage(h, topo_features, centers, log_sigmas,
                              rule_weights, rule_W, rule_b, cls_W, cls_b)
    return out, rules


# speedup vs baseline: 16.6316x; 16.6316x over previous
import functools

import jax
import jax.numpy as jnp
from jax import lax
from jax.experimental import pallas as pl
from jax.experimental.pallas import tpu as pltpu
from jax.experimental.pallas import tpu_sc as plsc

N = 10000
E = 160000
D_IN = 128
HID = 64
HEADS = 8
RULES = 10
OUT = 64
NEG = 0.2
EPS = 1e-5

ETOT = E + N
BLK = 128
NWORK = 32
EPAD = 172032
BLOCKS_TOTAL = EPAD // BLK
BLOCKS_PER_W = BLOCKS_TOTAL // NWORK
BLOCKS_PER_S = BLOCKS_TOTAL // 16

NB = 1000
GRID_N = N // NB


def _tc_front_body(x_ref, w1_ref, asm_ref, adm_ref,
                   h0_ref, h1_ref, h2_ref, h3_ref, ast_ref, adt_ref, gm_ref):
    i = pl.program_id(0)
    h = jnp.dot(x_ref[...], w1_ref[...], preferred_element_type=jnp.float32)
    h0_ref[...] = h[:, 0:128]
    h1_ref[...] = h[:, 128:256]
    h2_ref[...] = h[:, 256:384]
    h3_ref[...] = h[:, 384:512]
    asb = jnp.dot(h, asm_ref[...], preferred_element_type=jnp.float32)
    adb = jnp.dot(h, adm_ref[...], preferred_element_type=jnp.float32)
    ast_ref[...] = asb
    adt_ref[...] = adb
    cur = jnp.concatenate([jnp.max(asb, axis=0, keepdims=True),
                           jnp.max(adb, axis=0, keepdims=True)], axis=0)

    @pl.when(i == 0)
    def _():
        gm_ref[...] = cur

    @pl.when(i > 0)
    def _():
        gm_ref[...] = jnp.maximum(gm_ref[...], cur)


def _tc_front(x, W1, Asrc, Adst):
    plane = jax.ShapeDtypeStruct((N, 128), jnp.float32)
    return pl.pallas_call(
        _tc_front_body,
        grid=(GRID_N,),
        in_specs=[
            pl.BlockSpec((NB, D_IN), lambda i: (i, 0)),
            pl.BlockSpec((D_IN, HEADS * HID), lambda i: (0, 0)),
            pl.BlockSpec((HEADS * HID, 16), lambda i: (0, 0)),
            pl.BlockSpec((HEADS * HID, 16), lambda i: (0, 0)),
        ],
        out_specs=[
            pl.BlockSpec((NB, 128), lambda i: (i, 0)),
            pl.BlockSpec((NB, 128), lambda i: (i, 0)),
            pl.BlockSpec((NB, 128), lambda i: (i, 0)),
            pl.BlockSpec((NB, 128), lambda i: (i, 0)),
            pl.BlockSpec((NB, 16), lambda i: (i, 0)),
            pl.BlockSpec((NB, 16), lambda i: (i, 0)),
            pl.BlockSpec((2, 16), lambda i: (0, 0)),
        ],
        out_shape=[plane, plane, plane, plane,
                   jax.ShapeDtypeStruct((N, 16), jnp.float32),
                   jax.ShapeDtypeStruct((N, 16), jnp.float32),
                   jax.ShapeDtypeStruct((2, 16), jnp.float32)],
    )(x, W1, Asrc, Adst)


_MESH = plsc.VectorSubcoreMesh(core_axis_name="c", subcore_axis_name="s")


def _zero_rows(ref, ncols):
    z = jnp.zeros((16,), jnp.float32)

    def body(i, _):
        for j in range(ncols // 16):
            ref[i, pl.ds(j * 16, 16)] = z
        return 0

    lax.fori_loop(0, 128, body, 0, unroll=False)


def _zero_stripe(shared, zbuf, s):
    @pl.when(s < 15)
    def _():
        for m in range(5):
            pltpu.sync_copy(zbuf, shared.at[pl.ds(s * 640 + m * 128, 128)])

    @pl.when(s == 15)
    def _():
        for m in range(3):
            pltpu.sync_copy(zbuf, shared.at[pl.ds(9600 + m * 128, 128)])
        pltpu.sync_copy(zbuf.at[pl.ds(0, 16)], shared.at[pl.ds(9984, 16)])


def _copy_stripe(shared, dst, s):
    @pl.when(s < 15)
    def _():
        pltpu.sync_copy(shared.at[pl.ds(s * 640, 640)],
                        dst.at[pl.ds(s * 640, 640)])

    @pl.when(s == 15)
    def _():
        pltpu.sync_copy(shared.at[pl.ds(9600, 400)], dst.at[pl.ds(9600, 400)])


def _leaky(a):
    return jnp.where(a >= 0.0, a, NEG * a)


def _sc_e1_body(src_ref, dst_ref, ast_ref, adt_ref, gm_ref,
                ex_ref, den_ref,
                sidx, didx, gs, gd, exb, gmb, den_acc, sem):
    c = lax.axis_index("c")
    s = lax.axis_index("s")
    wid = s * 2 + c

    pltpu.sync_copy(gm_ref, gmb)
    g = _leaky(gmb[0] + gmb[1])

    _zero_rows(exb, 16)
    _zero_stripe(den_acc, exb, s)
    plsc.subcore_barrier()

    def blk(k, _):
        base = (wid * BLOCKS_PER_W + k) * BLK
        pltpu.sync_copy(src_ref.at[pl.ds(base, BLK)], sidx)
        pltpu.sync_copy(dst_ref.at[pl.ds(base, BLK)], didx)
        pltpu.async_copy(ast_ref.at[sidx], gs, sem).wait()
        pltpu.async_copy(adt_ref.at[didx], gd, sem).wait()

        def pe(e, _):
            a = _leaky(gs[e] + gd[e])
            exv = jnp.exp(a - g)
            okf = jnp.where(base + e < ETOT, 1.0, 0.0)
            exb[e] = exv * okf
            return 0

        lax.fori_loop(0, BLK, pe, 0, unroll=False)
        pltpu.sync_copy(exb, ex_ref.at[pl.ds(base, BLK)])
        pltpu.sync_copy(exb, den_acc.at[didx], add=True)
        return 0

    lax.fori_loop(0, BLOCKS_PER_W, blk, 0, unroll=False)
    plsc.subcore_barrier()

    @pl.when(c == 0)
    def _():
        _copy_stripe(den_acc, den_ref.at[0], s)

    @pl.when(c == 1)
    def _():
        _copy_stripe(den_acc, den_ref.at[1], s)


def _sc_e1(src, dst, ast, adt, gm):
    f = pl.kernel(
        _sc_e1_body,
        out_type=[jax.ShapeDtypeStruct((EPAD, 16), jnp.float32),
                  jax.ShapeDtypeStruct((2, N, 16), jnp.float32)],
        mesh=_MESH,
        compiler_params=pltpu.CompilerParams(use_tc_tiling_on_sc=False),
        scratch_types=[
            pltpu.VMEM((BLK,), jnp.int32),
            pltpu.VMEM((BLK,), jnp.int32),
            pltpu.VMEM((BLK, 16), jnp.float32),
            pltpu.VMEM((BLK, 16), jnp.float32),
            pltpu.VMEM((BLK, 16), jnp.float32),
            pltpu.VMEM((2, 16), jnp.float32),
            pltpu.VMEM_SHARED((N, 16), jnp.float32),
            pltpu.SemaphoreType.DMA,
        ],
    )
    return f(src, dst, ast, adt, gm)


def _sc_g1(src, dst, ex, hps):
    plane = jax.ShapeDtypeStruct((N, 128), jnp.float32)

    def body(src_ref, dst_ref, ex_ref, hp0, hp1, hp2, hp3,
             o0, o1, o2, o3, sidx, didx, exb, rowb, zb, acc, sem):
        c = lax.axis_index("c")
        s = lax.axis_index("s")

        _zero_rows(zb, 128)

        def do_plane(p, h_ref, o_ref):
            _zero_stripe(acc, zb, s)
            plsc.subcore_barrier()

            def blk(k, _):
                base = (s * BLOCKS_PER_S + k) * BLK
                pltpu.sync_copy(src_ref.at[pl.ds(base, BLK)], sidx)
                pltpu.sync_copy(dst_ref.at[pl.ds(base, BLK)], didx)
                pltpu.sync_copy(ex_ref.at[pl.ds(base, BLK)], exb)
                pltpu.async_copy(h_ref.at[sidx], rowb, sem).wait()

                def pe(e, _):
                    ev = exb[e]
                    a0 = ev[2 * p]
                    a1 = ev[2 * p + 1]
                    for j in range(8):
                        sl = pl.ds(j * 16, 16)
                        rowb[e, sl] = rowb[e, sl] * (a0 if j < 4 else a1)
                    return 0

                lax.fori_loop(0, BLK, pe, 0, unroll=False)
                pltpu.sync_copy(rowb, acc.at[didx], add=True)
                return 0

            lax.fori_loop(0, BLOCKS_PER_S, blk, 0, unroll=False)
            plsc.subcore_barrier()
            _copy_stripe(acc, o_ref, s)
            plsc.subcore_barrier()

        @pl.when(c == 0)
        def _():
            do_plane(0, hp0, o0)
            do_plane(1, hp1, o1)

        @pl.when(c == 1)
        def _():
            do_plane(2, hp2, o2)
            do_plane(3, hp3, o3)

    f = pl.kernel(
        body,
        out_type=[plane, plane, plane, plane],
        mesh=_MESH,
        compiler_params=pltpu.CompilerParams(use_tc_tiling_on_sc=False),
        scratch_types=[
            pltpu.VMEM((BLK,), jnp.int32),
            pltpu.VMEM((BLK,), jnp.int32),
            pltpu.VMEM((BLK, 16), jnp.float32),
            pltpu.VMEM((BLK, 128), jnp.float32),
            pltpu.VMEM((BLK, 128), jnp.float32),
            pltpu.VMEM_SHARED((N, 128), jnp.float32),
            pltpu.SemaphoreType.DMA,
        ],
    )
    return f(src, dst, ex, *hps)


def _sc_eg2(src, dst, a2st, a2dt, gm2, h2):
    def body(src_ref, dst_ref, ast_ref, adt_ref, gm_ref, h_ref,
             out_ref, den_ref,
             sidx, didx, gs, gd, exb, rowb, gmb, acc, den_acc, sem):
        c = lax.axis_index("c")
        s = lax.axis_index("s")
        wid = s * 2 + c

        pltpu.sync_copy(gm_ref, gmb)
        g = _leaky(gmb[0] + gmb[1])

        _zero_rows(exb, 16)
        _zero_rows(rowb, 64)
        _zero_stripe(den_acc, exb, s)
        _zero_stripe(acc, rowb, s)
        plsc.subcore_barrier()

        def blk(k, _):
            base = (wid * BLOCKS_PER_W + k) * BLK
            pltpu.sync_copy(src_ref.at[pl.ds(base, BLK)], sidx)
            pltpu.sync_copy(dst_ref.at[pl.ds(base, BLK)], didx)
            pltpu.async_copy(ast_ref.at[sidx], gs, sem).wait()
            pltpu.async_copy(adt_ref.at[didx], gd, sem).wait()
            pltpu.async_copy(h_ref.at[sidx], rowb, sem).wait()

            def pe(e, _):
                a = _leaky(gs[e] + gd[e])
                exv = jnp.exp(a - g)
                okf = jnp.where(base + e < ETOT, 1.0, 0.0)
                exv = exv * okf
                exb[e] = exv
                a0 = exv[0]
                for j in range(4):
                    sl = pl.ds(j * 16, 16)
                    rowb[e, sl] = rowb[e, sl] * a0
                return 0

            lax.fori_loop(0, BLK, pe, 0, unroll=False)
            pltpu.sync_copy(exb, den_acc.at[didx], add=True)
            pltpu.sync_copy(rowb, acc.at[didx], add=True)
            return 0

        lax.fori_loop(0, BLOCKS_PER_W, blk, 0, unroll=False)
        plsc.subcore_barrier()

        @pl.when(c == 0)
        def _():
            _copy_stripe(acc, out_ref.at[0], s)
            _copy_stripe(den_acc, den_ref.at[0], s)

        @pl.when(c == 1)
        def _():
            _copy_stripe(acc, out_ref.at[1], s)
            _copy_stripe(den_acc, den_ref.at[1], s)

    f = pl.kernel(
        body,
        out_type=[jax.ShapeDtypeStruct((2, N, HID), jnp.float32),
                  jax.ShapeDtypeStruct((2, N, 16), jnp.float32)],
        mesh=_MESH,
        compiler_params=pltpu.CompilerParams(use_tc_tiling_on_sc=False),
        scratch_types=[
            pltpu.VMEM((BLK,), jnp.int32),
            pltpu.VMEM((BLK,), jnp.int32),
            pltpu.VMEM((BLK, 16), jnp.float32),
            pltpu.VMEM((BLK, 16), jnp.float32),
            pltpu.VMEM((BLK, 16), jnp.float32),
            pltpu.VMEM((BLK, HID), jnp.float32),
            pltpu.VMEM((2, 16), jnp.float32),
            pltpu.VMEM_SHARED((N, HID), jnp.float32),
            pltpu.VMEM_SHARED((N, 16), jnp.float32),
            pltpu.SemaphoreType.DMA,
        ],
    )
    return f(src, dst, a2st, a2dt, gm2, h2)


def _tc_mid1_body(r0, r1, r2, r3, den_ref, b1_ref, h_ref, st_ref):
    i = pl.program_id(0)
    den = den_ref[0] + den_ref[1] + 1e-16
    parts = []
    for p, r in enumerate((r0, r1, r2, r3)):
        d0 = jnp.broadcast_to(den[:, 2 * p:2 * p + 1], (NB, HID))
        d1 = jnp.broadcast_to(den[:, 2 * p + 1:2 * p + 2], (NB, HID))
        parts.append(r[...] / jnp.concatenate([d0, d1], axis=1))
    hh = jnp.concatenate(parts, axis=1) + b1_ref[...]
    h_ref[...] = hh
    cur = jnp.concatenate([jnp.sum(hh, axis=0, keepdims=True),
                           jnp.sum(hh * hh, axis=0, keepdims=True)], axis=0)

    @pl.when(i == 0)
    def _():
        st_ref[...] = cur

    @pl.when(i > 0)
    def _():
        st_ref[...] = st_ref[...] + cur


def _tc_mid1(planes, den_parts, b1):
    return pl.pallas_call(
        _tc_mid1_body,
        grid=(GRID_N,),
        in_specs=[
            pl.BlockSpec((NB, 128), lambda i: (i, 0)),
            pl.BlockSpec((NB, 128), lambda i: (i, 0)),
            pl.BlockSpec((NB, 128), lambda i: (i, 0)),
            pl.BlockSpec((NB, 128), lambda i: (i, 0)),
            pl.BlockSpec((2, NB, 16), lambda i: (0, i, 0)),
            pl.BlockSpec((1, HEADS * HID), lambda i: (0, 0)),
        ],
        out_specs=[
            pl.BlockSpec((NB, HEADS * HID), lambda i: (i, 0)),
            pl.BlockSpec((2, HEADS * HID), lambda i: (0, 0)),
        ],
        out_shape=[jax.ShapeDtypeStruct((N, HEADS * HID), jnp.float32),
                   jax.ShapeDtypeStruct((2, HEADS * HID), jnp.float32)],
    )(*planes, den_parts, b1[None, :])


def _tc_mid2_body(h_ref, st_ref, g_ref, b_ref, w2_ref, a2s_ref, a2d_ref,
                  h2_ref, ast_ref, adt_ref, gm_ref):
    i = pl.program_id(0)
    mu = st_ref[0:1] / N
    var = st_ref[1:2] / N - mu * mu
    xn = g_ref[...] * (h_ref[...] - mu) / jnp.sqrt(var + EPS) + b_ref[...]
    el = jnp.where(xn > 0.0, xn, jnp.exp(xn) - 1.0)
    h2 = jnp.dot(el, w2_ref[...], preferred_element_type=jnp.float32)
    h2_ref[...] = h2
    asb = jnp.dot(h2, a2s_ref[...], preferred_element_type=jnp.float32)
    adb = jnp.dot(h2, a2d_ref[...], preferred_element_type=jnp.float32)
    ast_ref[...] = asb
    adt_ref[...] = adb
    cur = jnp.concatenate([jnp.max(asb, axis=0, keepdims=True),
                           jnp.max(adb, axis=0, keepdims=True)], axis=0)

    @pl.when(i == 0)
    def _():
        gm_ref[...] = cur

    @pl.when(i > 0)
    def _():
        gm_ref[...] = jnp.maximum(gm_ref[...], cur)


def _tc_mid2(h1out, stats, bn_g, bn_b, W2, A2s, A2d):
    return pl.pallas_call(
        _tc_mid2_body,
        grid=(GRID_N,),
        in_specs=[
            pl.BlockSpec((NB, HEADS * HID), lambda i: (i, 0)),
            pl.BlockSpec((2, HEADS * HID), lambda i: (0, 0)),
            pl.BlockSpec((1, HEADS * HID), lambda i: (0, 0)),
            pl.BlockSpec((1, HEADS * HID), lambda i: (0, 0)),
            pl.BlockSpec((HEADS * HID, HID), lambda i: (0, 0)),
            pl.BlockSpec((HID, 16), lambda i: (0, 0)),
            pl.BlockSpec((HID, 16), lambda i: (0, 0)),
        ],
        out_specs=[
            pl.BlockSpec((NB, HID), lambda i: (i, 0)),
            pl.BlockSpec((NB, 16), lambda i: (i, 0)),
            pl.BlockSpec((NB, 16), lambda i: (i, 0)),
            pl.BlockSpec((2, 16), lambda i: (0, 0)),
        ],
        out_shape=[jax.ShapeDtypeStruct((N, HID), jnp.float32),
                   jax.ShapeDtypeStruct((N, 16), jnp.float32),
                   jax.ShapeDtypeStruct((N, 16), jnp.float32),
                   jax.ShapeDtypeStruct((2, 16), jnp.float32)],
    )(h1out, stats, bn_g[None, :], bn_b[None, :], W2, A2s, A2d)


def _tc_final_body(o2_ref, d2_ref, b2_ref, topo_ref, centers_ref,
                   log_sigmas_ref, rule_w_ref, rule_W_ref, rule_b_ref,
                   cls_W_ref, cls_b_ref, out_ref, rules_ref):
    den = d2_ref[0, :, 0:1] + d2_ref[1, :, 0:1] + 1e-16
    h = (o2_ref[0] + o2_ref[1]) / jnp.broadcast_to(den, (NB, HID))
    h = h + b2_ref[...]

    topo = topo_ref[...]
    cc = centers_ref[...]
    q = 0.5 / (jnp.exp(log_sigmas_ref[...]) ** 2)
    A = jnp.dot(topo * topo, q.T, preferred_element_type=jnp.float32)
    B = jnp.dot(topo, (cc * q).T, preferred_element_type=jnp.float32)
    C = jnp.sum(cc * cc * q, axis=1)[None, :]
    logg = -(A - 2.0 * B + C)
    sig = 1.0 / (1.0 + jnp.exp(-rule_w_ref[...]))
    rules = jnp.exp(logg) * sig
    rules_ref[...] = rules

    rw = rule_W_ref[...]
    comb = (jnp.dot(h, rw[:HID], preferred_element_type=jnp.float32)
            + jnp.dot(rules, rw[HID:], preferred_element_type=jnp.float32)
            + rule_b_ref[...])
    h2 = jnp.maximum(comb, 0.0)
    o = jnp.dot(h2, cls_W_ref[...], preferred_element_type=jnp.float32) + cls_b_ref[...]
    m = jnp.max(o, axis=1, keepdims=True)
    lse = jnp.log(jnp.sum(jnp.exp(o - m), axis=1, keepdims=True)) + m
    out_ref[...] = o - lse


def _tc_final(out2_parts, den2_parts, b2, topo, centers, log_sigmas,
              rule_weights, rule_W, rule_b, cls_W, cls_b):
    return pl.pallas_call(
        _tc_final_body,
        grid=(GRID_N,),
        in_specs=[
            pl.BlockSpec((2, NB, HID), lambda i: (0, i, 0)),
            pl.BlockSpec((2, NB, 16), lambda i: (0, i, 0)),
            pl.BlockSpec((1, HID), lambda i: (0, 0)),
            pl.BlockSpec((NB, 6), lambda i: (i, 0)),
            pl.BlockSpec((RULES, 6), lambda i: (0, 0)),
            pl.BlockSpec((RULES, 6), lambda i: (0, 0)),
            pl.BlockSpec((1, RULES), lambda i: (0, 0)),
            pl.BlockSpec((HID + RULES, HID), lambda i: (0, 0)),
            pl.BlockSpec((1, HID), lambda i: (0, 0)),
            pl.BlockSpec((HID, OUT), lambda i: (0, 0)),
            pl.BlockSpec((1, OUT), lambda i: (0, 0)),
        ],
        out_specs=[
            pl.BlockSpec((NB, OUT), lambda i: (i, 0)),
            pl.BlockSpec((NB, RULES), lambda i: (i, 0)),
        ],
        out_shape=[jax.ShapeDtypeStruct((N, OUT), jnp.float32),
                   jax.ShapeDtypeStruct((N, RULES), jnp.float32)],
    )(out2_parts, den2_parts, b2[None, :], topo, centers, log_sigmas,
      rule_weights[None, :], rule_W, rule_b[None, :], cls_W, cls_b[None, :])


def kernel(x, edge_index, topo_features, W1, a_src1, a_dst1, b1, bn_g, bn_b,
           W2, a_src2, a_dst2, b2, centers, log_sigmas, rule_weights, rule_W,
           rule_b, cls_W, cls_b):
    loop = jnp.arange(N, dtype=edge_index.dtype)
    pad = jnp.zeros((EPAD - ETOT,), dtype=edge_index.dtype)
    src = jnp.concatenate([edge_index[0], loop, pad]).astype(jnp.int32)
    dst = jnp.concatenate([edge_index[1], loop, pad]).astype(jnp.int32)

    headmask = (jnp.arange(HEADS)[:, None] ==
                (jnp.arange(16) % HEADS)[None, :]).astype(jnp.float32)
    Asrc = (a_src1[:, :, None] * headmask[:, None, :]).reshape(HEADS * HID, 16)
    Adst = (a_dst1[:, :, None] * headmask[:, None, :]).reshape(HEADS * HID, 16)
    A2s = jnp.broadcast_to(a_src2[0][:, None], (HID, 16))
    A2d = jnp.broadcast_to(a_dst2[0][:, None], (HID, 16))

    hp0, hp1, hp2, hp3, ast, adt, gm1 = _tc_front(x, W1, Asrc, Adst)
    ex, den_parts = _sc_e1(src, dst, ast, adt, gm1)
    o0, o1, o2, o3 = _sc_g1(src, dst, ex, (hp0, hp1, hp2, hp3))
    h1out, stats = _tc_mid1((o0, o1, o2, o3), den_parts, b1)
    h2, a2st, a2dt, gm2 = _tc_mid2(h1out, stats, bn_g, bn_b, W2, A2s, A2d)
    out2_parts, den2_parts = _sc_eg2(src, dst, a2st, a2dt, gm2, h2)
    out, rules = _tc_final(out2_parts, den2_parts, b2, topo_features,
                           centers, log_sigmas, rule_weights, rule_W, rule_b,
                           cls_W, cls_b)
    return out, rules


# speedup vs baseline: 28.9621x; 1.7414x over previous
import functools

import jax
import jax.numpy as jnp
from jax import lax
from jax.experimental import pallas as pl
from jax.experimental.pallas import tpu as pltpu
from jax.experimental.pallas import tpu_sc as plsc

N = 10000
E = 160000
D_IN = 128
HID = 64
HEADS = 8
RULES = 10
OUT = 64
NEG = 0.2
EPS = 1e-5

ETOT = E + N
BLK = 128
NWORK = 32
EPAD = 172032
BLOCKS_TOTAL = EPAD // BLK
BLOCKS_PER_W = BLOCKS_TOTAL // NWORK
BLOCKS_PER_S = BLOCKS_TOTAL // 16

NB = 1000
GRID_N = N // NB


def _tc_front_body(x_ref, w1_ref, asm_ref, adm_ref,
                   h0_ref, h1_ref, h2_ref, h3_ref, ast_ref, adt_ref, gm_ref):
    i = pl.program_id(0)
    h = jnp.dot(x_ref[...], w1_ref[...], preferred_element_type=jnp.float32)
    h0_ref[...] = h[:, 0:128]
    h1_ref[...] = h[:, 128:256]
    h2_ref[...] = h[:, 256:384]
    h3_ref[...] = h[:, 384:512]
    asb = jnp.dot(h, asm_ref[...], preferred_element_type=jnp.float32)
    adb = jnp.dot(h, adm_ref[...], preferred_element_type=jnp.float32)
    ast_ref[...] = asb
    adt_ref[...] = adb
    cur = jnp.concatenate([jnp.max(asb, axis=0, keepdims=True),
                           jnp.max(adb, axis=0, keepdims=True)], axis=0)

    @pl.when(i == 0)
    def _():
        gm_ref[...] = cur

    @pl.when(i > 0)
    def _():
        gm_ref[...] = jnp.maximum(gm_ref[...], cur)


def _tc_front(x, W1, Asrc, Adst):
    plane = jax.ShapeDtypeStruct((N, 128), jnp.float32)
    return pl.pallas_call(
        _tc_front_body,
        grid=(GRID_N,),
        in_specs=[
            pl.BlockSpec((NB, D_IN), lambda i: (i, 0)),
            pl.BlockSpec((D_IN, HEADS * HID), lambda i: (0, 0)),
            pl.BlockSpec((HEADS * HID, 16), lambda i: (0, 0)),
            pl.BlockSpec((HEADS * HID, 16), lambda i: (0, 0)),
        ],
        out_specs=[
            pl.BlockSpec((NB, 128), lambda i: (i, 0)),
            pl.BlockSpec((NB, 128), lambda i: (i, 0)),
            pl.BlockSpec((NB, 128), lambda i: (i, 0)),
            pl.BlockSpec((NB, 128), lambda i: (i, 0)),
            pl.BlockSpec((NB, 16), lambda i: (i, 0)),
            pl.BlockSpec((NB, 16), lambda i: (i, 0)),
            pl.BlockSpec((2, 16), lambda i: (0, 0)),
        ],
        out_shape=[plane, plane, plane, plane,
                   jax.ShapeDtypeStruct((N, 16), jnp.float32),
                   jax.ShapeDtypeStruct((N, 16), jnp.float32),
                   jax.ShapeDtypeStruct((2, 16), jnp.float32)],
    )(x, W1, Asrc, Adst)


_MESH = plsc.VectorSubcoreMesh(core_axis_name="c", subcore_axis_name="s")


def _zero_rows(ref, ncols):
    z = jnp.zeros((16,), jnp.float32)

    def body(i, _):
        for j in range(ncols // 16):
            ref[i, pl.ds(j * 16, 16)] = z
        return 0

    lax.fori_loop(0, 128, body, 0, unroll=False)


def _zero_stripe(shared, zbuf, s):
    @pl.when(s < 15)
    def _():
        for m in range(5):
            pltpu.sync_copy(zbuf, shared.at[pl.ds(s * 640 + m * 128, 128)])

    @pl.when(s == 15)
    def _():
        for m in range(3):
            pltpu.sync_copy(zbuf, shared.at[pl.ds(9600 + m * 128, 128)])
        pltpu.sync_copy(zbuf.at[pl.ds(0, 16)], shared.at[pl.ds(9984, 16)])


def _copy_stripe(shared, dst, s):
    @pl.when(s < 15)
    def _():
        pltpu.sync_copy(shared.at[pl.ds(s * 640, 640)],
                        dst.at[pl.ds(s * 640, 640)])

    @pl.when(s == 15)
    def _():
        pltpu.sync_copy(shared.at[pl.ds(9600, 400)], dst.at[pl.ds(9600, 400)])


def _leaky(a):
    return jnp.where(a >= 0.0, a, NEG * a)


def _sc_e1_body(src_ref, dst_ref, ast_ref, adt_ref, gm_ref,
                ex_ref, den_ref,
                sidxa, didxa, gsA, gdA, gsB, gdB, exb, gmb, den_acc,
                semA, semB):
    c = lax.axis_index("c")
    s = lax.axis_index("s")
    wid = s * 2 + c
    b0 = wid * BLOCKS_PER_W

    pltpu.sync_copy(gm_ref, gmb)
    g = _leaky(gmb[0] + gmb[1])

    _zero_rows(exb, 16)
    _zero_stripe(den_acc, exb, s)
    pltpu.sync_copy(src_ref.at[pl.ds(b0, BLOCKS_PER_W)], sidxa)
    pltpu.sync_copy(dst_ref.at[pl.ds(b0, BLOCKS_PER_W)], didxa)
    plsc.subcore_barrier()

    def start(k, gs, gd, sem):
        pltpu.async_copy(ast_ref.at[sidxa.at[k]], gs, sem)
        pltpu.async_copy(adt_ref.at[didxa.at[k]], gd, sem)

    def drain(gs, gd, sem):
        pltpu.make_async_copy(ast_ref.at[sidxa.at[0]], gs, sem).wait()
        pltpu.make_async_copy(adt_ref.at[sidxa.at[0]], gd, sem).wait()

    def compute(k, gs, gd):
        base = (b0 + k) * BLK

        def pe(e, _):
            a = _leaky(gs[e] + gd[e])
            exv = jnp.exp(a - g)
            okf = jnp.where(base + e < ETOT, 1.0, 0.0)
            exb[e] = exv * okf
            return 0

        lax.fori_loop(0, BLK, pe, 0, unroll=4)
        pltpu.sync_copy(exb, ex_ref.at[pl.ds(base, BLK)])
        pltpu.sync_copy(exb, den_acc.at[didxa.at[k]], add=True)

    start(0, gsA, gdA, semA)

    def pair(t, _):
        k0 = 2 * t
        start(k0 + 1, gsB, gdB, semB)
        drain(gsA, gdA, semA)
        compute(k0, gsA, gdA)

        @pl.when(t < BLOCKS_PER_W // 2 - 1)
        def _():
            start(k0 + 2, gsA, gdA, semA)

        drain(gsB, gdB, semB)
        compute(k0 + 1, gsB, gdB)
        return 0

    lax.fori_loop(0, BLOCKS_PER_W // 2, pair, 0, unroll=False)
    plsc.subcore_barrier()

    @pl.when(c == 0)
    def _():
        _copy_stripe(den_acc, den_ref.at[0], s)

    @pl.when(c == 1)
    def _():
        _copy_stripe(den_acc, den_ref.at[1], s)


def _sc_e1(src2d, dst2d, ast, adt, gm):
    f = pl.kernel(
        _sc_e1_body,
        out_type=[jax.ShapeDtypeStruct((EPAD, 16), jnp.float32),
                  jax.ShapeDtypeStruct((2, N, 16), jnp.float32)],
        mesh=_MESH,
        compiler_params=pltpu.CompilerParams(use_tc_tiling_on_sc=False),
        scratch_types=[
            pltpu.VMEM((BLOCKS_PER_W, BLK), jnp.int32),
            pltpu.VMEM((BLOCKS_PER_W, BLK), jnp.int32),
            pltpu.VMEM((BLK, 16), jnp.float32),
            pltpu.VMEM((BLK, 16), jnp.float32),
            pltpu.VMEM((BLK, 16), jnp.float32),
            pltpu.VMEM((BLK, 16), jnp.float32),
            pltpu.VMEM((BLK, 16), jnp.float32),
            pltpu.VMEM((2, 16), jnp.float32),
            pltpu.VMEM_SHARED((N, 16), jnp.float32),
            pltpu.SemaphoreType.DMA,
            pltpu.SemaphoreType.DMA,
        ],
    )
    return f(src2d, dst2d, ast, adt, gm)


SEG = 42


def _sc_g1(src2d, dst2d, ex, hps):
    plane = jax.ShapeDtypeStruct((N, 128), jnp.float32)

    def body(src_ref, dst_ref, ex_ref, hp0, hp1, hp2, hp3,
             o0, o1, o2, o3,
             sidxa, didxa, exbA, exbB, rowbA, rowbB, acc,
             gsemA, gsemB):
        c = lax.axis_index("c")
        s = lax.axis_index("s")
        b0 = s * BLOCKS_PER_S

        def do_plane(p, h_ref, o_ref):
            _zero_rows(rowbA, 128)
            _zero_stripe(acc, rowbA, s)
            plsc.subcore_barrier()

            for seg in range(BLOCKS_PER_S // SEG):
                sb = b0 + seg * SEG
                pltpu.sync_copy(src_ref.at[pl.ds(sb, SEG)], sidxa)
                pltpu.sync_copy(dst_ref.at[pl.ds(sb, SEG)], didxa)

                def start(k, rowb, exbuf, sem):
                    pltpu.async_copy(h_ref.at[sidxa.at[k]], rowb, sem)
                    pltpu.async_copy(ex_ref.at[pl.ds((sb + k) * BLK, BLK)],
                                     exbuf, sem)

                def drain(rowb, exbuf, sem):
                    pltpu.make_async_copy(h_ref.at[sidxa.at[0]], rowb,
                                          sem).wait()
                    pltpu.make_async_copy(ex_ref.at[pl.ds(0, BLK)], exbuf,
                                          sem).wait()

                def compute(k, rowb, exbuf):
                    def pe(e, _):
                        ev = exbuf[e]
                        a0 = ev[2 * p]
                        a1 = ev[2 * p + 1]
                        for j in range(8):
                            sl = pl.ds(j * 16, 16)
                            rowb[e, sl] = rowb[e, sl] * (a0 if j < 4 else a1)
                        return 0

                    lax.fori_loop(0, BLK, pe, 0, unroll=4)
                    pltpu.sync_copy(rowb, acc.at[didxa.at[k]], add=True)

                start(0, rowbA, exbA, gsemA)

                def pair(t, _):
                    k0 = 2 * t
                    start(k0 + 1, rowbB, exbB, gsemB)
                    drain(rowbA, exbA, gsemA)
                    compute(k0, rowbA, exbA)

                    @pl.when(t < SEG // 2 - 1)
                    def _():
                        start(k0 + 2, rowbA, exbA, gsemA)

                    drain(rowbB, exbB, gsemB)
                    compute(k0 + 1, rowbB, exbB)
                    return 0

                lax.fori_loop(0, SEG // 2, pair, 0, unroll=False)

            plsc.subcore_barrier()
            _copy_stripe(acc, o_ref, s)
            plsc.subcore_barrier()

        @pl.when(c == 0)
        def _():
            do_plane(0, hp0, o0)
            do_plane(1, hp1, o1)

        @pl.when(c == 1)
        def _():
            do_plane(2, hp2, o2)
            do_plane(3, hp3, o3)

    f = pl.kernel(
        body,
        out_type=[plane, plane, plane, plane],
        mesh=_MESH,
        compiler_params=pltpu.CompilerParams(use_tc_tiling_on_sc=False),
        scratch_types=[
            pltpu.VMEM((SEG, BLK), jnp.int32),
            pltpu.VMEM((SEG, BLK), jnp.int32),
            pltpu.VMEM((BLK, 16), jnp.float32),
            pltpu.VMEM((BLK, 16), jnp.float32),
            pltpu.VMEM((BLK, 128), jnp.float32),
            pltpu.VMEM((BLK, 128), jnp.float32),
            pltpu.VMEM_SHARED((N, 128), jnp.float32),
            pltpu.SemaphoreType.DMA,
            pltpu.SemaphoreType.DMA,
        ],
    )
    return f(src2d, dst2d, ex, *hps)


def _sc_eg2(src2d, dst2d, a2st, a2dt, gm2, h2):
    def body(src_ref, dst_ref, ast_ref, adt_ref, gm_ref, h_ref,
             out_ref, den_ref,
             sidxa, didxa, gsA, gdA, gsB, gdB, rowbA, rowbB, exb, gmb,
             acc, den_acc, semA, semB):
        c = lax.axis_index("c")
        s = lax.axis_index("s")
        wid = s * 2 + c
        b0 = wid * BLOCKS_PER_W

        pltpu.sync_copy(gm_ref, gmb)
        g = _leaky(gmb[0] + gmb[1])

        _zero_rows(exb, 16)
        _zero_rows(rowbA, HID)
        _zero_stripe(den_acc, exb, s)
        _zero_stripe(acc, rowbA, s)
        pltpu.sync_copy(src_ref.at[pl.ds(b0, BLOCKS_PER_W)], sidxa)
        pltpu.sync_copy(dst_ref.at[pl.ds(b0, BLOCKS_PER_W)], didxa)
        plsc.subcore_barrier()

        def start(k, gs, gd, rowb, sem):
            pltpu.async_copy(ast_ref.at[sidxa.at[k]], gs, sem)
            pltpu.async_copy(adt_ref.at[didxa.at[k]], gd, sem)
            pltpu.async_copy(h_ref.at[sidxa.at[k]], rowb, sem)

        def drain(gs, gd, rowb, sem):
            pltpu.make_async_copy(ast_ref.at[sidxa.at[0]], gs, sem).wait()
            pltpu.make_async_copy(adt_ref.at[sidxa.at[0]], gd, sem).wait()
            pltpu.make_async_copy(h_ref.at[sidxa.at[0]], rowb, sem).wait()

        def compute(k, gs, gd, rowb):
            base = (b0 + k) * BLK

            def pe(e, _):
                a = _leaky(gs[e] + gd[e])
                exv = jnp.exp(a - g)
                okf = jnp.where(base + e < ETOT, 1.0, 0.0)
                exv = exv * okf
                exb[e] = exv
                a0 = exv[0]
                for j in range(4):
                    sl = pl.ds(j * 16, 16)
                    rowb[e, sl] = rowb[e, sl] * a0
                return 0

            lax.fori_loop(0, BLK, pe, 0, unroll=4)
            pltpu.sync_copy(exb, den_acc.at[didxa.at[k]], add=True)
            pltpu.sync_copy(rowb, acc.at[didxa.at[k]], add=True)

        start(0, gsA, gdA, rowbA, semA)

        def pair(t, _):
            k0 = 2 * t
            start(k0 + 1, gsB, gdB, rowbB, semB)
            drain(gsA, gdA, rowbA, semA)
            compute(k0, gsA, gdA, rowbA)

            @pl.when(t < BLOCKS_PER_W // 2 - 1)
            def _():
                start(k0 + 2, gsA, gdA, rowbA, semA)

            drain(gsB, gdB, rowbB, semB)
            compute(k0 + 1, gsB, gdB, rowbB)
            return 0

        lax.fori_loop(0, BLOCKS_PER_W // 2, pair, 0, unroll=False)
        plsc.subcore_barrier()

        @pl.when(c == 0)
        def _():
            _copy_stripe(acc, out_ref.at[0], s)
            _copy_stripe(den_acc, den_ref.at[0], s)

        @pl.when(c == 1)
        def _():
            _copy_stripe(acc, out_ref.at[1], s)
            _copy_stripe(den_acc, den_ref.at[1], s)

    f = pl.kernel(
        body,
        out_type=[jax.ShapeDtypeStruct((2, N, HID), jnp.float32),
                  jax.ShapeDtypeStruct((2, N, 16), jnp.float32)],
        mesh=_MESH,
        compiler_params=pltpu.CompilerParams(use_tc_tiling_on_sc=False),
        scratch_types=[
            pltpu.VMEM((BLOCKS_PER_W, BLK), jnp.int32),
            pltpu.VMEM((BLOCKS_PER_W, BLK), jnp.int32),
            pltpu.VMEM((BLK, 16), jnp.float32),
            pltpu.VMEM((BLK, 16), jnp.float32),
            pltpu.VMEM((BLK, 16), jnp.float32),
            pltpu.VMEM((BLK, 16), jnp.float32),
            pltpu.VMEM((BLK, HID), jnp.float32),
            pltpu.VMEM((BLK, HID), jnp.float32),
            pltpu.VMEM((BLK, 16), jnp.float32),
            pltpu.VMEM((2, 16), jnp.float32),
            pltpu.VMEM_SHARED((N, HID), jnp.float32),
            pltpu.VMEM_SHARED((N, 16), jnp.float32),
            pltpu.SemaphoreType.DMA,
            pltpu.SemaphoreType.DMA,
        ],
    )
    return f(src2d, dst2d, a2st, a2dt, gm2, h2)


def _tc_mid1_body(r0, r1, r2, r3, den_ref, b1_ref, h_ref, st_ref):
    i = pl.program_id(0)
    den = den_ref[0] + den_ref[1] + 1e-16
    parts = []
    for p, r in enumerate((r0, r1, r2, r3)):
        d0 = jnp.broadcast_to(den[:, 2 * p:2 * p + 1], (NB, HID))
        d1 = jnp.broadcast_to(den[:, 2 * p + 1:2 * p + 2], (NB, HID))
        parts.append(r[...] / jnp.concatenate([d0, d1], axis=1))
    hh = jnp.concatenate(parts, axis=1) + b1_ref[...]
    h_ref[...] = hh
    cur = jnp.concatenate([jnp.sum(hh, axis=0, keepdims=True),
                           jnp.sum(hh * hh, axis=0, keepdims=True)], axis=0)

    @pl.when(i == 0)
    def _():
        st_ref[...] = cur

    @pl.when(i > 0)
    def _():
        st_ref[...] = st_ref[...] + cur


def _tc_mid1(planes, den_parts, b1):
    return pl.pallas_call(
        _tc_mid1_body,
        grid=(GRID_N,),
        in_specs=[
            pl.BlockSpec((NB, 128), lambda i: (i, 0)),
            pl.BlockSpec((NB, 128), lambda i: (i, 0)),
            pl.BlockSpec((NB, 128), lambda i: (i, 0)),
            pl.BlockSpec((NB, 128), lambda i: (i, 0)),
            pl.BlockSpec((2, NB, 16), lambda i: (0, i, 0)),
            pl.BlockSpec((1, HEADS * HID), lambda i: (0, 0)),
        ],
        out_specs=[
            pl.BlockSpec((NB, HEADS * HID), lambda i: (i, 0)),
            pl.BlockSpec((2, HEADS * HID), lambda i: (0, 0)),
        ],
        out_shape=[jax.ShapeDtypeStruct((N, HEADS * HID), jnp.float32),
                   jax.ShapeDtypeStruct((2, HEADS * HID), jnp.float32)],
    )(*planes, den_parts, b1[None, :])


def _tc_mid2_body(h_ref, st_ref, g_ref, b_ref, w2_ref, a2s_ref, a2d_ref,
                  h2_ref, ast_ref, adt_ref, gm_ref):
    i = pl.program_id(0)
    mu = st_ref[0:1] / N
    var = st_ref[1:2] / N - mu * mu
    xn = g_ref[...] * (h_ref[...] - mu) / jnp.sqrt(var + EPS) + b_ref[...]
    el = jnp.where(xn > 0.0, xn, jnp.exp(xn) - 1.0)
    h2 = jnp.dot(el, w2_ref[...], preferred_element_type=jnp.float32)
    h2_ref[...] = h2
    asb = jnp.dot(h2, a2s_ref[...], preferred_element_type=jnp.float32)
    adb = jnp.dot(h2, a2d_ref[...], preferred_element_type=jnp.float32)
    ast_ref[...] = asb
    adt_ref[...] = adb
    cur = jnp.concatenate([jnp.max(asb, axis=0, keepdims=True),
                           jnp.max(adb, axis=0, keepdims=True)], axis=0)

    @pl.when(i == 0)
    def _():
        gm_ref[...] = cur

    @pl.when(i > 0)
    def _():
        gm_ref[...] = jnp.maximum(gm_ref[...], cur)


def _tc_mid2(h1out, stats, bn_g, bn_b, W2, A2s, A2d):
    return pl.pallas_call(
        _tc_mid2_body,
        grid=(GRID_N,),
        in_specs=[
            pl.BlockSpec((NB, HEADS * HID), lambda i: (i, 0)),
            pl.BlockSpec((2, HEADS * HID), lambda i: (0, 0)),
            pl.BlockSpec((1, HEADS * HID), lambda i: (0, 0)),
            pl.BlockSpec((1, HEADS * HID), lambda i: (0, 0)),
            pl.BlockSpec((HEADS * HID, HID), lambda i: (0, 0)),
            pl.BlockSpec((HID, 16), lambda i: (0, 0)),
            pl.BlockSpec((HID, 16), lambda i: (0, 0)),
        ],
        out_specs=[
            pl.BlockSpec((NB, HID), lambda i: (i, 0)),
            pl.BlockSpec((NB, 16), lambda i: (i, 0)),
            pl.BlockSpec((NB, 16), lambda i: (i, 0)),
            pl.BlockSpec((2, 16), lambda i: (0, 0)),
        ],
        out_shape=[jax.ShapeDtypeStruct((N, HID), jnp.float32),
                   jax.ShapeDtypeStruct((N, 16), jnp.float32),
                   jax.ShapeDtypeStruct((N, 16), jnp.float32),
                   jax.ShapeDtypeStruct((2, 16), jnp.float32)],
    )(h1out, stats, bn_g[None, :], bn_b[None, :], W2, A2s, A2d)


def _tc_final_body(o2_ref, d2_ref, b2_ref, topo_ref, centers_ref,
                   log_sigmas_ref, rule_w_ref, rule_W_ref, rule_b_ref,
                   cls_W_ref, cls_b_ref, out_ref, rules_ref):
    den = d2_ref[0, :, 0:1] + d2_ref[1, :, 0:1] + 1e-16
    h = (o2_ref[0] + o2_ref[1]) / jnp.broadcast_to(den, (NB, HID))
    h = h + b2_ref[...]

    topo = topo_ref[...]
    cc = centers_ref[...]
    q = 0.5 / (jnp.exp(log_sigmas_ref[...]) ** 2)
    A = jnp.dot(topo * topo, q.T, preferred_element_type=jnp.float32)
    B = jnp.dot(topo, (cc * q).T, preferred_element_type=jnp.float32)
    C = jnp.sum(cc * cc * q, axis=1)[None, :]
    logg = -(A - 2.0 * B + C)
    sig = 1.0 / (1.0 + jnp.exp(-rule_w_ref[...]))
    rules = jnp.exp(logg) * sig
    rules_ref[...] = rules

    rw = rule_W_ref[...]
    comb = (jnp.dot(h, rw[:HID], preferred_element_type=jnp.float32)
            + jnp.dot(rules, rw[HID:], preferred_element_type=jnp.float32)
            + rule_b_ref[...])
    h2 = jnp.maximum(comb, 0.0)
    o = jnp.dot(h2, cls_W_ref[...], preferred_element_type=jnp.float32) + cls_b_ref[...]
    m = jnp.max(o, axis=1, keepdims=True)
    lse = jnp.log(jnp.sum(jnp.exp(o - m), axis=1, keepdims=True)) + m
    out_ref[...] = o - lse


def _tc_final(out2_parts, den2_parts, b2, topo, centers, log_sigmas,
              rule_weights, rule_W, rule_b, cls_W, cls_b):
    return pl.pallas_call(
        _tc_final_body,
        grid=(GRID_N,),
        in_specs=[
            pl.BlockSpec((2, NB, HID), lambda i: (0, i, 0)),
            pl.BlockSpec((2, NB, 16), lambda i: (0, i, 0)),
            pl.BlockSpec((1, HID), lambda i: (0, 0)),
            pl.BlockSpec((NB, 6), lambda i: (i, 0)),
            pl.BlockSpec((RULES, 6), lambda i: (0, 0)),
            pl.BlockSpec((RULES, 6), lambda i: (0, 0)),
            pl.BlockSpec((1, RULES), lambda i: (0, 0)),
            pl.BlockSpec((HID + RULES, HID), lambda i: (0, 0)),
            pl.BlockSpec((1, HID), lambda i: (0, 0)),
            pl.BlockSpec((HID, OUT), lambda i: (0, 0)),
            pl.BlockSpec((1, OUT), lambda i: (0, 0)),
        ],
        out_specs=[
            pl.BlockSpec((NB, OUT), lambda i: (i, 0)),
            pl.BlockSpec((NB, RULES), lambda i: (i, 0)),
        ],
        out_shape=[jax.ShapeDtypeStruct((N, OUT), jnp.float32),
                   jax.ShapeDtypeStruct((N, RULES), jnp.float32)],
    )(out2_parts, den2_parts, b2[None, :], topo, centers, log_sigmas,
      rule_weights[None, :], rule_W, rule_b[None, :], cls_W, cls_b[None, :])


def kernel(x, edge_index, topo_features, W1, a_src1, a_dst1, b1, bn_g, bn_b,
           W2, a_src2, a_dst2, b2, centers, log_sigmas, rule_weights, rule_W,
           rule_b, cls_W, cls_b):
    loop = jnp.arange(N, dtype=edge_index.dtype)
    pad = jnp.zeros((EPAD - ETOT,), dtype=edge_index.dtype)
    src = jnp.concatenate([edge_index[0], loop, pad]).astype(jnp.int32)
    dst = jnp.concatenate([edge_index[1], loop, pad]).astype(jnp.int32)
    src2d = src.reshape(BLOCKS_TOTAL, BLK)
    dst2d = dst.reshape(BLOCKS_TOTAL, BLK)

    headmask = (jnp.arange(HEADS)[:, None] ==
                (jnp.arange(16) % HEADS)[None, :]).astype(jnp.float32)
    Asrc = (a_src1[:, :, None] * headmask[:, None, :]).reshape(HEADS * HID, 16)
    Adst = (a_dst1[:, :, None] * headmask[:, None, :]).reshape(HEADS * HID, 16)
    A2s = jnp.broadcast_to(a_src2[0][:, None], (HID, 16))
    A2d = jnp.broadcast_to(a_dst2[0][:, None], (HID, 16))

    hp0, hp1, hp2, hp3, ast, adt, gm1 = _tc_front(x, W1, Asrc, Adst)
    ex, den_parts = _sc_e1(src2d, dst2d, ast, adt, gm1)
    o0, o1, o2, o3 = _sc_g1(src2d, dst2d, ex, (hp0, hp1, hp2, hp3))
    h1out, stats = _tc_mid1((o0, o1, o2, o3), den_parts, b1)
    h2, a2st, a2dt, gm2 = _tc_mid2(h1out, stats, bn_g, bn_b, W2, A2s, A2d)
    out2_parts, den2_parts = _sc_eg2(src2d, dst2d, a2st, a2dt, gm2, h2)
    out, rules = _tc_final(out2_parts, den2_parts, b2, topo_features,
                           centers, log_sigmas, rule_weights, rule_W, rule_b,
                           cls_W, cls_b)
    return out, rules
